# Initial kernel scaffold; baseline (speedup 1.0000x reference)
import jax
import jax.numpy as jnp
from jax.experimental import pallas as pl


def kernel(x, edge_index, W1, att_src1, att_dst1, b1, W2, att_src2, att_dst2, b2, Wo, bo):
    raise NotImplementedError("write your pallas kernel here")


# speedup vs baseline: 10.5971x; 10.5971x over previous
import jax
import jax.numpy as jnp
from jax import lax
from jax.experimental import pallas as pl
from jax.experimental.pallas import tpu as pltpu
from jax.experimental.pallas import tpu_sc as plsc

N_NODES = 10000
N_EDGES = 320000
D_IN = 128
D_HID = 128
HEADS = 8
DH = HEADS * D_HID

NC, NS = 2, 16
NW = NC * NS
NP = 10240
EW = N_EDGES // NW
CA = 80
CB = 16
BN = 1000

_f32 = jnp.float32


def _lin_att_body(x_ref, w_ref, ac_ref, h_ref, ap_ref):
    h = jnp.dot(x_ref[...], w_ref[...], preferred_element_type=_f32)
    h_ref[...] = h
    ap_ref[...] = jnp.dot(h, ac_ref[...], preferred_element_type=_f32)


def _mid_body(mp_ref, b_ref, w_ref, ac_ref, h_ref, ap_ref):
    t = (mp_ref[0] + mp_ref[1]) * (1.0 / HEADS) + b_ref[...]
    t = jnp.where(t > 0.0, t, jnp.exp(t) - 1.0)
    h = jnp.dot(t, w_ref[...], preferred_element_type=_f32)
    h_ref[...] = h
    ap_ref[...] = jnp.dot(h, ac_ref[...], preferred_element_type=_f32)


def _out_body(mp_ref, b_ref, wo_ref, bo_ref, o_ref):
    t = (mp_ref[0] + mp_ref[1]) * (1.0 / HEADS) + b_ref[...]
    t = jnp.where(t > 0.0, t, jnp.exp(t) - 1.0)
    o_ref[...] = jnp.dot(t, wo_ref[...], preferred_element_type=_f32) + bo_ref[...]


def _inv_body(d_ref, o_ref):
    o_ref[...] = 1.0 / (d_ref[0] + d_ref[1] + 1e-16)


def _phase_a(acat, srcs, dsts, z128, ex_out, den_out,
             sid_v, did_v, a_v, b_v, exf_v, exs_v, den_sh, sem):
    cid = lax.axis_index("c")
    sid = lax.axis_index("s")
    wid = sid * NC + cid
    nz = NP // NS
    pltpu.sync_copy(z128.at[pl.ds(sid * nz, nz), :], den_sh.at[pl.ds(sid * nz, nz), :])
    pltpu.sync_copy(z128.at[pl.ds(0, CA), :], exs_v)
    plsc.subcore_barrier()
    ebase = wid * EW

    def chunk(k, carry):
        base = ebase + k * CA
        pltpu.sync_copy(srcs.at[pl.ds(base, CA)], sid_v)
        pltpu.sync_copy(dsts.at[pl.ds(base, CA)], did_v)
        pltpu.async_copy(acat.at[sid_v], a_v, sem).wait()
        pltpu.async_copy(acat.at[did_v], b_v, sem).wait()

        def row(i, c2):
            v = a_v[i, pl.ds(0, 16)] + b_v[i, pl.ds(16, 16)]
            v = jnp.where(v >= 0.0, v, 0.2 * v)
            ex = jnp.exp(v)
            exf_v[pl.ds(i * 16, 16)] = ex
            exs_v[i, pl.ds(0, 16)] = ex
            return c2

        lax.fori_loop(0, CA, row, 0)
        pltpu.sync_copy(exf_v, ex_out.at[pl.ds(base * 16, CA * 16)])
        pltpu.sync_copy(exs_v, den_sh.at[did_v], add=True)
        return carry

    lax.fori_loop(0, EW // CA, chunk, 0)
    plsc.subcore_barrier()
    rb = sid * nz
    pltpu.sync_copy(den_sh.at[pl.ds(rb, nz), :], den_out.at[cid, pl.ds(rb, nz), :])


def _phase_b(h_t, ex_t, inv_t, srcs, dsts, z128, out_p,
             sid_v, did_v, h_v, exf_v, inv_v, m_v, acc_sh, sem):
    cid = lax.axis_index("c")
    sid = lax.axis_index("s")
    wid = sid * NC + cid
    nz = NP // NS
    pltpu.sync_copy(z128.at[pl.ds(sid * nz, nz), :], acc_sh.at[pl.ds(sid * nz, nz), :])
    plsc.subcore_barrier()
    ebase = wid * EW

    def chunk(k, carry):
        base = ebase + k * CB
        pltpu.sync_copy(srcs.at[pl.ds(base, CB)], sid_v)
        pltpu.sync_copy(dsts.at[pl.ds(base, CB)], did_v)
        pltpu.async_copy(h_t.at[sid_v], h_v, sem).wait()
        pltpu.async_copy(inv_t.at[did_v], inv_v, sem).wait()
        pltpu.sync_copy(ex_t.at[pl.ds(base * 16, CB * 16)], exf_v)

        def edge(i, c2):
            att = exf_v[pl.ds(i * 16, 16)] * inv_v[i, pl.ds(0, 16)]
            a = [att[h] for h in range(HEADS)]
            for j in range(D_HID // 16):
                acc = a[0] * h_v[i, pl.ds(j * 16, 16)]
                for h in range(1, HEADS):
                    acc = acc + a[h] * h_v[i, pl.ds(h * D_HID + j * 16, 16)]
                m_v[i, pl.ds(j * 16, 16)] = acc
            return c2

        lax.fori_loop(0, CB, edge, 0)
        pltpu.sync_copy(m_v, acc_sh.at[did_v], add=True)
        return carry

    lax.fori_loop(0, EW // CB, chunk, 0)
    plsc.subcore_barrier()
    rb = sid * nz
    pltpu.sync_copy(acc_sh.at[pl.ds(rb, nz), :], out_p.at[cid, pl.ds(rb, nz), :])


def _acat(att_s, att_d):
    eye = jnp.eye(HEADS, dtype=_f32)
    a_s = (att_s.reshape(HEADS, D_HID, 1) * eye[:, None, :]).reshape(DH, HEADS)
    a_d = (att_d.reshape(HEADS, D_HID, 1) * eye[:, None, :]).reshape(DH, HEADS)
    z8 = jnp.zeros((DH, 8), _f32)
    z96 = jnp.zeros((DH, 96), _f32)
    return jnp.concatenate([a_s, z8, a_d, z96], axis=1)


def _sc_mesh():
    return plsc.VectorSubcoreMesh(core_axis_name="c", subcore_axis_name="s")


def _gat_sc_layer(h, acat_tab, src, dst, z128):
    ex, den = pl.kernel(
        _phase_a,
        out_type=[jax.ShapeDtypeStruct((N_EDGES * 16,), _f32),
                  jax.ShapeDtypeStruct((NC, NP, 128), _f32)],
        mesh=_sc_mesh(),
        scratch_types=[pltpu.VMEM((CA,), jnp.int32), pltpu.VMEM((CA,), jnp.int32),
                       pltpu.VMEM((CA, 128), _f32), pltpu.VMEM((CA, 128), _f32),
                       pltpu.VMEM((CA * 16,), _f32), pltpu.VMEM((CA, 128), _f32),
                       pltpu.VMEM_SHARED((NP, 128), _f32),
                       pltpu.SemaphoreType.DMA],
    )(acat_tab, src, dst, z128)

    inv = pl.pallas_call(
        _inv_body,
        grid=(10,),
        in_specs=[pl.BlockSpec((NC, NP // 10, 128), lambda i: (0, i, 0))],
        out_specs=pl.BlockSpec((NP // 10, 128), lambda i: (i, 0)),
        out_shape=jax.ShapeDtypeStruct((NP, 128), _f32),
    )(den)

    msg = pl.kernel(
        _phase_b,
        out_type=jax.ShapeDtypeStruct((NC, NP, D_HID), _f32),
        mesh=_sc_mesh(),
        scratch_types=[pltpu.VMEM((CB,), jnp.int32), pltpu.VMEM((CB,), jnp.int32),
                       pltpu.VMEM((CB, DH), _f32), pltpu.VMEM((CB * 16,), _f32),
                       pltpu.VMEM((CB, 128), _f32), pltpu.VMEM((CB, D_HID), _f32),
                       pltpu.VMEM_SHARED((NP, D_HID), _f32),
                       pltpu.SemaphoreType.DMA],
    )(h, ex, inv, src, dst, z128)
    return msg


def kernel(x, edge_index, W1, att_src1, att_dst1, b1, W2, att_src2, att_dst2, b2, Wo, bo):
    src = edge_index[0].astype(jnp.int32)
    dst = edge_index[1].astype(jnp.int32)
    z128 = jnp.zeros((NP, D_HID), _f32)
    grid = (N_NODES // BN,)

    h1, ap1 = pl.pallas_call(
        _lin_att_body,
        grid=grid,
        in_specs=[pl.BlockSpec((BN, D_IN), lambda i: (i, 0)),
                  pl.BlockSpec((D_IN, DH), lambda i: (0, 0)),
                  pl.BlockSpec((DH, 128), lambda i: (0, 0))],
        out_specs=[pl.BlockSpec((BN, DH), lambda i: (i, 0)),
                   pl.BlockSpec((BN, 128), lambda i: (i, 0))],
        out_shape=[jax.ShapeDtypeStruct((N_NODES, DH), _f32),
                   jax.ShapeDtypeStruct((N_NODES, 128), _f32)],
    )(x, W1, _acat(att_src1, att_dst1))

    msg1 = _gat_sc_layer(h1, ap1, src, dst, z128)

    h2, ap2 = pl.pallas_call(
        _mid_body,
        grid=grid,
        in_specs=[pl.BlockSpec((NC, BN, D_HID), lambda i: (0, i, 0)),
                  pl.BlockSpec((1, D_HID), lambda i: (0, 0)),
                  pl.BlockSpec((D_HID, DH), lambda i: (0, 0)),
                  pl.BlockSpec((DH, 128), lambda i: (0, 0))],
        out_specs=[pl.BlockSpec((BN, DH), lambda i: (i, 0)),
                   pl.BlockSpec((BN, 128), lambda i: (i, 0))],
        out_shape=[jax.ShapeDtypeStruct((N_NODES, DH), _f32),
                   jax.ShapeDtypeStruct((N_NODES, 128), _f32)],
    )(msg1, b1.reshape(1, D_HID), W2, _acat(att_src2, att_dst2))

    msg2 = _gat_sc_layer(h2, ap2, src, dst, z128)

    out = pl.pallas_call(
        _out_body,
        grid=grid,
        in_specs=[pl.BlockSpec((NC, BN, D_HID), lambda i: (0, i, 0)),
                  pl.BlockSpec((1, D_HID), lambda i: (0, 0)),
                  pl.BlockSpec((D_HID, D_HID), lambda i: (0, 0)),
                  pl.BlockSpec((1, D_HID), lambda i: (0, 0))],
        out_specs=pl.BlockSpec((BN, D_HID), lambda i: (i, 0)),
        out_shape=jax.ShapeDtypeStruct((N_NODES, D_HID), _f32),
    )(msg2, b2.reshape(1, D_HID), Wo, bo.reshape(1, D_HID))
    return out


# speedup vs baseline: 23.1124x; 2.1810x over previous
import jax
import jax.numpy as jnp
from jax import lax
from jax.experimental import pallas as pl
from jax.experimental.pallas import tpu as pltpu
from jax.experimental.pallas import tpu_sc as plsc

N_NODES = 10000
N_EDGES = 320000
D_IN = 128
D_HID = 128
HEADS = 8
DH = HEADS * D_HID

NC, NS = 2, 16
NW = NC * NS
NP = 10240
EW = N_EDGES // NW
CA = 80
CB = 16
BN = 1000

_f32 = jnp.float32


def _lin_att_body(x_ref, w_ref, ac_ref, h_ref, ap_ref):
    h = jnp.dot(x_ref[...], w_ref[...], preferred_element_type=_f32)
    h_ref[...] = h
    ap_ref[...] = jnp.dot(h, ac_ref[...], preferred_element_type=_f32)


def _mid_body(mp_ref, b_ref, w_ref, ac_ref, h_ref, ap_ref):
    t = (mp_ref[0] + mp_ref[1]) * (1.0 / HEADS) + b_ref[...]
    t = jnp.where(t > 0.0, t, jnp.exp(t) - 1.0)
    h = jnp.dot(t, w_ref[...], preferred_element_type=_f32)
    h_ref[...] = h
    ap_ref[...] = jnp.dot(h, ac_ref[...], preferred_element_type=_f32)


def _out_body(mp_ref, b_ref, wo_ref, bo_ref, o_ref):
    t = (mp_ref[0] + mp_ref[1]) * (1.0 / HEADS) + b_ref[...]
    t = jnp.where(t > 0.0, t, jnp.exp(t) - 1.0)
    o_ref[...] = jnp.dot(t, wo_ref[...], preferred_element_type=_f32) + bo_ref[...]


def _inv_body(d_ref, o_ref):
    o_ref[...] = 1.0 / (d_ref[0] + d_ref[1] + 1e-16)


def _phase_a(acat, srcs, dsts, z128, ex_out, den_out,
             sid_v, did_v, a_v, b_v, exf_v, exs_v, den_sh, sem):
    cid = lax.axis_index("c")
    sid = lax.axis_index("s")
    wid = sid * NC + cid
    nz = NP // NS
    pltpu.sync_copy(z128.at[pl.ds(sid * nz, nz), :], den_sh.at[pl.ds(sid * nz, nz), :])
    pltpu.sync_copy(z128.at[pl.ds(0, CA), :], exs_v)
    plsc.subcore_barrier()
    ebase = wid * EW

    def chunk(k, carry):
        base = ebase + k * CA
        pltpu.sync_copy(srcs.at[pl.ds(base, CA)], sid_v)
        pltpu.sync_copy(dsts.at[pl.ds(base, CA)], did_v)
        pltpu.async_copy(acat.at[sid_v], a_v, sem).wait()
        pltpu.async_copy(acat.at[did_v], b_v, sem).wait()

        def row(i, c2):
            v = a_v[i, pl.ds(0, 16)] + b_v[i, pl.ds(16, 16)]
            v = jnp.where(v >= 0.0, v, 0.2 * v)
            ex = jnp.exp(v)
            exf_v[pl.ds(i * 16, 16)] = ex
            exs_v[i, pl.ds(0, 16)] = ex
            return c2

        lax.fori_loop(0, CA, row, 0)
        pltpu.sync_copy(exf_v, ex_out.at[pl.ds(base * 16, CA * 16)])
        pltpu.sync_copy(exs_v, den_sh.at[did_v], add=True)
        return carry

    lax.fori_loop(0, EW // CA, chunk, 0)
    plsc.subcore_barrier()
    rb = sid * nz
    pltpu.sync_copy(den_sh.at[pl.ds(rb, nz), :], den_out.at[cid, pl.ds(rb, nz), :])


NCHB = EW // CB
SC_CH = 5
NSC = NCHB // SC_CH


def _phase_b(h_t, ex_t, inv_t, srcs4, dsts4, z128, out_p,
             sid_s, did_s, h_v, inv_v, exc_v, m_v, acc_sh, sg, ss, si):
    cid = lax.axis_index("c")
    sid = lax.axis_index("s")
    wid = sid * NC + cid
    nz = NP // NS
    pltpu.sync_copy(z128.at[pl.ds(sid * nz, nz), :], acc_sh.at[pl.ds(sid * nz, nz), :])
    plsc.subcore_barrier()
    ebase = wid * EW

    def _ids(s):
        t = s % 3
        return (
            pltpu.make_async_copy(srcs4.at[wid, s], sid_s.at[t], si.at[t]),
            pltpu.make_async_copy(dsts4.at[wid, s], did_s.at[t], si.at[t]),
        )

    def _gathers(c):
        p = c % 2
        t = (c // SC_CH) % 3
        l = c % SC_CH
        return (
            pltpu.make_async_copy(h_t.at[sid_s.at[t, l]], h_v.at[p], sg.at[p]),
            pltpu.make_async_copy(inv_t.at[did_s.at[t, l]], inv_v.at[p], sg.at[p]),
            pltpu.make_async_copy(ex_t.at[pl.ds((ebase + c * CB) * 16, CB * 16)],
                                  exc_v.at[p], sg.at[p]),
        )

    def _issue(c):
        for d in _gathers(c):
            d.start()

    def _waitg(c):
        for d in _gathers(c):
            d.wait()

    def _scatter(c):
        p = c % 2
        t = (c // SC_CH) % 3
        l = c % SC_CH
        return pltpu.make_async_copy(m_v.at[p], acc_sh.at[did_s.at[t, l]], ss.at[p])

    pltpu.sync_copy(srcs4.at[wid, 0], sid_s.at[0])
    pltpu.sync_copy(dsts4.at[wid, 0], did_s.at[0])
    for d in _ids(1):
        d.start()
    _issue(0)

    def step(c, carry):
        p = c % 2

        @pl.when(c + 1 < NCHB)
        def _():
            @pl.when((c + 1) % SC_CH == 0)
            def _():
                s1 = (c + 1) // SC_CH
                for d in _ids(s1):
                    d.wait()

                @pl.when(s1 + 1 < NSC)
                def _():
                    for d in _ids(s1 + 1):
                        d.start()

            _issue(c + 1)

        _waitg(c)

        @pl.when(c >= 2)
        def _():
            _scatter(c - 2).wait()

        def edge(i, c2):
            att = exc_v[p, pl.ds(i * 16, 16)] * inv_v[p, i, pl.ds(0, 16)]
            a = [att[h] for h in range(HEADS)]
            for j in range(D_HID // 16):
                acc = a[0] * h_v[p, i, pl.ds(j * 16, 16)]
                for h in range(1, HEADS):
                    acc = acc + a[h] * h_v[p, i, pl.ds(h * D_HID + j * 16, 16)]
                m_v[p, i, pl.ds(j * 16, 16)] = acc
            return c2

        lax.fori_loop(0, CB, edge, 0)
        _scatter(c).start(add=True)
        return carry

    lax.fori_loop(0, NCHB, step, 0)
    _scatter(NCHB - 2).wait()
    _scatter(NCHB - 1).wait()
    plsc.subcore_barrier()
    rb = sid * nz
    pltpu.sync_copy(acc_sh.at[pl.ds(rb, nz), :], out_p.at[cid, pl.ds(rb, nz), :])


def _acat(att_s, att_d):
    eye = jnp.eye(HEADS, dtype=_f32)
    a_s = (att_s.reshape(HEADS, D_HID, 1) * eye[:, None, :]).reshape(DH, HEADS)
    a_d = (att_d.reshape(HEADS, D_HID, 1) * eye[:, None, :]).reshape(DH, HEADS)
    z8 = jnp.zeros((DH, 8), _f32)
    z96 = jnp.zeros((DH, 96), _f32)
    return jnp.concatenate([a_s, z8, a_d, z96], axis=1)


def _sc_mesh():
    return plsc.VectorSubcoreMesh(core_axis_name="c", subcore_axis_name="s")


def _gat_sc_layer(h, acat_tab, src, dst, src4, dst4, z128):
    ex, den = pl.kernel(
        _phase_a,
        out_type=[jax.ShapeDtypeStruct((N_EDGES * 16,), _f32),
                  jax.ShapeDtypeStruct((NC, NP, 128), _f32)],
        mesh=_sc_mesh(),
        scratch_types=[pltpu.VMEM((CA,), jnp.int32), pltpu.VMEM((CA,), jnp.int32),
                       pltpu.VMEM((CA, 128), _f32), pltpu.VMEM((CA, 128), _f32),
                       pltpu.VMEM((CA * 16,), _f32), pltpu.VMEM((CA, 128), _f32),
                       pltpu.VMEM_SHARED((NP, 128), _f32),
                       pltpu.SemaphoreType.DMA],
    )(acat_tab, src, dst, z128)

    inv = pl.pallas_call(
        _inv_body,
        grid=(10,),
        in_specs=[pl.BlockSpec((NC, NP // 10, 128), lambda i: (0, i, 0))],
        out_specs=pl.BlockSpec((NP // 10, 128), lambda i: (i, 0)),
        out_shape=jax.ShapeDtypeStruct((NP, 128), _f32),
    )(den)

    msg = pl.kernel(
        _phase_b,
        out_type=jax.ShapeDtypeStruct((NC, NP, D_HID), _f32),
        mesh=_sc_mesh(),
        scratch_types=[pltpu.VMEM((3, SC_CH, CB), jnp.int32),
                       pltpu.VMEM((3, SC_CH, CB), jnp.int32),
                       pltpu.VMEM((2, CB, DH), _f32),
                       pltpu.VMEM((2, CB, 128), _f32),
                       pltpu.VMEM((2, CB * 16), _f32),
                       pltpu.VMEM((2, CB, D_HID), _f32),
                       pltpu.VMEM_SHARED((NP, D_HID), _f32),
                       pltpu.SemaphoreType.DMA((2,)),
                       pltpu.SemaphoreType.DMA((2,)),
                       pltpu.SemaphoreType.DMA((3,))],
    )(h, ex, inv, src4, dst4, z128)
    return msg


def kernel(x, edge_index, W1, att_src1, att_dst1, b1, W2, att_src2, att_dst2, b2, Wo, bo):
    src = edge_index[0].astype(jnp.int32)
    dst = edge_index[1].astype(jnp.int32)
    src4 = src.reshape(NW, NSC, SC_CH, CB)
    dst4 = dst.reshape(NW, NSC, SC_CH, CB)
    z128 = jnp.zeros((NP, D_HID), _f32)
    grid = (N_NODES // BN,)

    h1, ap1 = pl.pallas_call(
        _lin_att_body,
        grid=grid,
        in_specs=[pl.BlockSpec((BN, D_IN), lambda i: (i, 0)),
                  pl.BlockSpec((D_IN, DH), lambda i: (0, 0)),
                  pl.BlockSpec((DH, 128), lambda i: (0, 0))],
        out_specs=[pl.BlockSpec((BN, DH), lambda i: (i, 0)),
                   pl.BlockSpec((BN, 128), lambda i: (i, 0))],
        out_shape=[jax.ShapeDtypeStruct((N_NODES, DH), _f32),
                   jax.ShapeDtypeStruct((N_NODES, 128), _f32)],
    )(x, W1, _acat(att_src1, att_dst1))

    msg1 = _gat_sc_layer(h1, ap1, src, dst, src4, dst4, z128)

    h2, ap2 = pl.pallas_call(
        _mid_body,
        grid=grid,
        in_specs=[pl.BlockSpec((NC, BN, D_HID), lambda i: (0, i, 0)),
                  pl.BlockSpec((1, D_HID), lambda i: (0, 0)),
                  pl.BlockSpec((D_HID, DH), lambda i: (0, 0)),
                  pl.BlockSpec((DH, 128), lambda i: (0, 0))],
        out_specs=[pl.BlockSpec((BN, DH), lambda i: (i, 0)),
                   pl.BlockSpec((BN, 128), lambda i: (i, 0))],
        out_shape=[jax.ShapeDtypeStruct((N_NODES, DH), _f32),
                   jax.ShapeDtypeStruct((N_NODES, 128), _f32)],
    )(msg1, b1.reshape(1, D_HID), W2, _acat(att_src2, att_dst2))

    msg2 = _gat_sc_layer(h2, ap2, src, dst, src4, dst4, z128)

    out = pl.pallas_call(
        _out_body,
        grid=grid,
        in_specs=[pl.BlockSpec((NC, BN, D_HID), lambda i: (0, i, 0)),
                  pl.BlockSpec((1, D_HID), lambda i: (0, 0)),
                  pl.BlockSpec((D_HID, D_HID), lambda i: (0, 0)),
                  pl.BlockSpec((1, D_HID), lambda i: (0, 0))],
        out_specs=pl.BlockSpec((BN, D_HID), lambda i: (i, 0)),
        out_shape=jax.ShapeDtypeStruct((N_NODES, D_HID), _f32),
    )(msg2, b2.reshape(1, D_HID), Wo, bo.reshape(1, D_HID))
    return out


# speedup vs baseline: 24.6704x; 1.0674x over previous
import jax
import jax.numpy as jnp
from jax import lax
from jax.experimental import pallas as pl
from jax.experimental.pallas import tpu as pltpu
from jax.experimental.pallas import tpu_sc as plsc

N_NODES = 10000
N_EDGES = 320000
D_IN = 128
D_HID = 128
HEADS = 8
DH = HEADS * D_HID

NC, NS = 2, 16
NW = NC * NS
NP = 10240
EW = N_EDGES // NW
CA = 80
CB = 16
BN = 1000

_f32 = jnp.float32


def _lin_att_body(x_ref, w_ref, ac_ref, h_ref, ap_ref):
    h = jnp.dot(x_ref[...], w_ref[...], preferred_element_type=_f32)
    h_ref[...] = h
    ap_ref[...] = jnp.dot(h, ac_ref[...], preferred_element_type=_f32)


def _mid_body(mp_ref, b_ref, w_ref, ac_ref, h_ref, ap_ref):
    t = (mp_ref[0] + mp_ref[1]) * (1.0 / HEADS) + b_ref[...]
    t = jnp.where(t > 0.0, t, jnp.exp(t) - 1.0)
    h = jnp.dot(t, w_ref[...], preferred_element_type=_f32)
    h_ref[...] = h
    ap_ref[...] = jnp.dot(h, ac_ref[...], preferred_element_type=_f32)


def _out_body(mp_ref, b_ref, wo_ref, bo_ref, o_ref):
    t = (mp_ref[0] + mp_ref[1]) * (1.0 / HEADS) + b_ref[...]
    t = jnp.where(t > 0.0, t, jnp.exp(t) - 1.0)
    o_ref[...] = jnp.dot(t, wo_ref[...], preferred_element_type=_f32) + bo_ref[...]


def _inv_body(d_ref, o_ref):
    o_ref[...] = 1.0 / (d_ref[0] + d_ref[1] + 1e-16)


NCHA = EW // CA
SC_A = 5
NSCA = NCHA // SC_A


def _phase_a(acat, srcs4, dsts4, z16, ex_out, den_out,
             sid_s, did_s, a_v, b_v, exf_v, exs_v, den_sh, sem):
    cid = lax.axis_index("c")
    sid = lax.axis_index("s")
    wid = sid * NC + cid
    nz = NP // NS
    pltpu.sync_copy(z16.at[pl.ds(sid * nz, nz), :], den_sh.at[pl.ds(sid * nz, nz), :])
    plsc.subcore_barrier()
    ebase = wid * EW

    def superchunk(s, carry):
        pltpu.sync_copy(srcs4.at[wid, s], sid_s)
        pltpu.sync_copy(dsts4.at[wid, s], did_s)

        def chunk(l, c2):
            pltpu.async_copy(acat.at[sid_s.at[l]], a_v, sem).wait()
            pltpu.async_copy(acat.at[did_s.at[l]], b_v, sem).wait()

            def row(i, c3):
                v = a_v[i, pl.ds(0, 16)] + b_v[i, pl.ds(16, 16)]
                v = jnp.where(v >= 0.0, v, 0.2 * v)
                ex = jnp.exp(v)
                exf_v[pl.ds((l * CA + i) * 16, 16)] = ex
                exs_v[i] = ex
                return c3

            lax.fori_loop(0, CA, row, 0)
            pltpu.sync_copy(exs_v, den_sh.at[did_s.at[l]], add=True)
            return c2

        lax.fori_loop(0, SC_A, chunk, 0)
        pltpu.sync_copy(exf_v, ex_out.at[pl.ds((ebase + s * SC_A * CA) * 16,
                                               SC_A * CA * 16)])
        return carry

    lax.fori_loop(0, NSCA, superchunk, 0)
    plsc.subcore_barrier()
    rb = sid * nz
    pltpu.sync_copy(den_sh.at[pl.ds(rb, nz), :], den_out.at[cid, pl.ds(rb, nz), :])


NCHB = EW // CB
SC_CH = 5
NSC = NCHB // SC_CH


def _phase_b(h_t, ex_t, inv_t, srcs4, dsts4, z128, out_p,
             sid_s, did_s, h_v, inv_v, exc_v, m_v, acc_sh, sg, ss, si):
    cid = lax.axis_index("c")
    sid = lax.axis_index("s")
    wid = sid * NC + cid
    nz = NP // NS
    pltpu.sync_copy(z128.at[pl.ds(sid * nz, nz), :], acc_sh.at[pl.ds(sid * nz, nz), :])
    plsc.subcore_barrier()
    ebase = wid * EW

    def _ids(s):
        t = s % 3
        return (
            pltpu.make_async_copy(srcs4.at[wid, s], sid_s.at[t], si.at[t]),
            pltpu.make_async_copy(dsts4.at[wid, s], did_s.at[t], si.at[t]),
        )

    def _gathers(c):
        p = c % 2
        t = (c // SC_CH) % 3
        l = c % SC_CH
        return (
            pltpu.make_async_copy(h_t.at[sid_s.at[t, l]], h_v.at[p], sg.at[p]),
            pltpu.make_async_copy(inv_t.at[did_s.at[t, l]], inv_v.at[p], sg.at[p]),
            pltpu.make_async_copy(ex_t.at[pl.ds((ebase + c * CB) * 16, CB * 16)],
                                  exc_v.at[p], sg.at[p]),
        )

    def _issue(c):
        for d in _gathers(c):
            d.start()

    def _waitg(c):
        for d in _gathers(c):
            d.wait()

    def _scatter(c):
        p = c % 2
        t = (c // SC_CH) % 3
        l = c % SC_CH
        return pltpu.make_async_copy(m_v.at[p], acc_sh.at[did_s.at[t, l]], ss.at[p])

    pltpu.sync_copy(srcs4.at[wid, 0], sid_s.at[0])
    pltpu.sync_copy(dsts4.at[wid, 0], did_s.at[0])
    for d in _ids(1):
        d.start()
    _issue(0)

    def step(c, carry):
        p = c % 2

        @pl.when(c + 1 < NCHB)
        def _():
            @pl.when((c + 1) % SC_CH == 0)
            def _():
                s1 = (c + 1) // SC_CH
                for d in _ids(s1):
                    d.wait()

                @pl.when(s1 + 1 < NSC)
                def _():
                    for d in _ids(s1 + 1):
                        d.start()

            _issue(c + 1)

        _waitg(c)

        @pl.when(c >= 2)
        def _():
            _scatter(c - 2).wait()

        def edge(i, c2):
            att = exc_v[p, pl.ds(i * 16, 16)] * inv_v[p, i, pl.ds(0, 16)]
            a = [att[h] for h in range(HEADS)]
            for j in range(D_HID // 16):
                acc = a[0] * h_v[p, i, pl.ds(j * 16, 16)]
                for h in range(1, HEADS):
                    acc = acc + a[h] * h_v[p, i, pl.ds(h * D_HID + j * 16, 16)]
                m_v[p, i, pl.ds(j * 16, 16)] = acc
            return c2

        lax.fori_loop(0, CB, edge, 0)
        _scatter(c).start(add=True)
        return carry

    lax.fori_loop(0, NCHB, step, 0)
    _scatter(NCHB - 2).wait()
    _scatter(NCHB - 1).wait()
    plsc.subcore_barrier()
    rb = sid * nz
    pltpu.sync_copy(acc_sh.at[pl.ds(rb, nz), :], out_p.at[cid, pl.ds(rb, nz), :])


def _acat(att_s, att_d):
    eye = jnp.eye(HEADS, dtype=_f32)
    a_s = (att_s.reshape(HEADS, D_HID, 1) * eye[:, None, :]).reshape(DH, HEADS)
    a_d = (att_d.reshape(HEADS, D_HID, 1) * eye[:, None, :]).reshape(DH, HEADS)
    z8 = jnp.zeros((DH, 8), _f32)
    z96 = jnp.zeros((DH, 96), _f32)
    return jnp.concatenate([a_s, z8, a_d, z96], axis=1)


def _sc_mesh():
    return plsc.VectorSubcoreMesh(core_axis_name="c", subcore_axis_name="s")


def _gat_sc_layer(h, ap, src4a, dst4a, src4, dst4, z16, z128):
    ex, den = pl.kernel(
        _phase_a,
        out_type=[jax.ShapeDtypeStruct((N_EDGES * 16,), _f32),
                  jax.ShapeDtypeStruct((NC, NP, 16), _f32)],
        mesh=_sc_mesh(),
        scratch_types=[pltpu.VMEM((SC_A, CA), jnp.int32),
                       pltpu.VMEM((SC_A, CA), jnp.int32),
                       pltpu.VMEM((CA, 128), _f32), pltpu.VMEM((CA, 128), _f32),
                       pltpu.VMEM((SC_A * CA * 16,), _f32),
                       pltpu.VMEM((CA, 16), _f32),
                       pltpu.VMEM_SHARED((NP, 16), _f32),
                       pltpu.SemaphoreType.DMA],
    )(ap, src4a, dst4a, z16)

    inv16 = pl.pallas_call(
        _inv_body,
        grid=(10,),
        in_specs=[pl.BlockSpec((NC, NP * 16 // 1280, 128), lambda i: (0, i, 0))],
        out_specs=pl.BlockSpec((NP * 16 // 1280, 128), lambda i: (i, 0)),
        out_shape=jax.ShapeDtypeStruct((1280, 128), _f32),
    )(den.reshape(NC, 1280, 128)).reshape(NP, 16)
    inv = jnp.pad(inv16, ((0, 0), (0, 112)))

    msg = pl.kernel(
        _phase_b,
        out_type=jax.ShapeDtypeStruct((NC, NP, D_HID), _f32),
        mesh=_sc_mesh(),
        scratch_types=[pltpu.VMEM((3, SC_CH, CB), jnp.int32),
                       pltpu.VMEM((3, SC_CH, CB), jnp.int32),
                       pltpu.VMEM((2, CB, DH), _f32),
                       pltpu.VMEM((2, CB, 128), _f32),
                       pltpu.VMEM((2, CB * 16), _f32),
                       pltpu.VMEM((2, CB, D_HID), _f32),
                       pltpu.VMEM_SHARED((NP, D_HID), _f32),
                       pltpu.SemaphoreType.DMA((2,)),
                       pltpu.SemaphoreType.DMA((2,)),
                       pltpu.SemaphoreType.DMA((3,))],
    )(h, ex, inv, src4, dst4, z128)
    return msg


def kernel(x, edge_index, W1, att_src1, att_dst1, b1, W2, att_src2, att_dst2, b2, Wo, bo):
    src = edge_index[0].astype(jnp.int32)
    dst = edge_index[1].astype(jnp.int32)
    src4 = src.reshape(NW, NSC, SC_CH, CB)
    dst4 = dst.reshape(NW, NSC, SC_CH, CB)
    src4a = src.reshape(NW, NSCA, SC_A, CA)
    dst4a = dst.reshape(NW, NSCA, SC_A, CA)
    z16 = jnp.zeros((NP, 16), _f32)
    z128 = jnp.zeros((NP, D_HID), _f32)
    grid = (N_NODES // BN,)

    h1, ap1 = pl.pallas_call(
        _lin_att_body,
        grid=grid,
        in_specs=[pl.BlockSpec((BN, D_IN), lambda i: (i, 0)),
                  pl.BlockSpec((D_IN, DH), lambda i: (0, 0)),
                  pl.BlockSpec((DH, 128), lambda i: (0, 0))],
        out_specs=[pl.BlockSpec((BN, DH), lambda i: (i, 0)),
                   pl.BlockSpec((BN, 128), lambda i: (i, 0))],
        out_shape=[jax.ShapeDtypeStruct((N_NODES, DH), _f32),
                   jax.ShapeDtypeStruct((N_NODES, 128), _f32)],
    )(x, W1, _acat(att_src1, att_dst1))

    msg1 = _gat_sc_layer(h1, ap1, src4a, dst4a, src4, dst4, z16, z128)

    h2, ap2 = pl.pallas_call(
        _mid_body,
        grid=grid,
        in_specs=[pl.BlockSpec((NC, BN, D_HID), lambda i: (0, i, 0)),
                  pl.BlockSpec((1, D_HID), lambda i: (0, 0)),
                  pl.BlockSpec((D_HID, DH), lambda i: (0, 0)),
                  pl.BlockSpec((DH, 128), lambda i: (0, 0))],
        out_specs=[pl.BlockSpec((BN, DH), lambda i: (i, 0)),
                   pl.BlockSpec((BN, 128), lambda i: (i, 0))],
        out_shape=[jax.ShapeDtypeStruct((N_NODES, DH), _f32),
                   jax.ShapeDtypeStruct((N_NODES, 128), _f32)],
    )(msg1, b1.reshape(1, D_HID), W2, _acat(att_src2, att_dst2))

    msg2 = _gat_sc_layer(h2, ap2, src4a, dst4a, src4, dst4, z16, z128)

    out = pl.pallas_call(
        _out_body,
        grid=grid,
        in_specs=[pl.BlockSpec((NC, BN, D_HID), lambda i: (0, i, 0)),
                  pl.BlockSpec((1, D_HID), lambda i: (0, 0)),
                  pl.BlockSpec((D_HID, D_HID), lambda i: (0, 0)),
                  pl.BlockSpec((1, D_HID), lambda i: (0, 0))],
        out_specs=pl.BlockSpec((BN, D_HID), lambda i: (i, 0)),
        out_shape=jax.ShapeDtypeStruct((N_NODES, D_HID), _f32),
    )(msg2, b2.reshape(1, D_HID), Wo, bo.reshape(1, D_HID))
    return out
